# Initial kernel scaffold; baseline (speedup 1.0000x reference)
import jax
import jax.numpy as jnp
from jax.experimental import pallas as pl


def kernel(text, emb_table, fc_w, fc_b):
    raise NotImplementedError("write your pallas kernel here")


# speedup vs baseline: 132.9783x; 132.9783x over previous
import functools

import jax
import jax.numpy as jnp
from jax import lax
from jax.experimental import pallas as pl
from jax.experimental.pallas import tpu as pltpu
from jax.experimental.pallas import tpu_sc as plsc

_NUM_EMB = 10000
_EMB_DIM = 128
_NUM_CLASSES = 3
_B = 16384
_L = 200

_LANES = 16
_L_PAD = 208
_V_PAD = 10016
_NUM_WORKERS = 32
_ROWS_PER_W = _B // _NUM_WORKERS
_CHUNK = 64
_NCHUNK = _ROWS_PER_W // _CHUNK


def _project_body(w_ref, emb_ref, out_ref):
    out_ref[...] = lax.dot_general(
        w_ref[...], emb_ref[...], (((1,), (1,)), ((), ())),
        preferred_element_type=jnp.float32)


def _project(fc_w_pad, emb_table):
    return pl.pallas_call(
        _project_body,
        out_shape=jax.ShapeDtypeStruct((8, _NUM_EMB), jnp.float32),
    )(fc_w_pad, emb_table)


_mesh = plsc.VectorSubcoreMesh(core_axis_name="c", subcore_axis_name="s")


@functools.partial(
    pl.kernel,
    out_type=jax.ShapeDtypeStruct((_B, _LANES), jnp.float32),
    mesh=_mesh,
    scratch_types=[
        pltpu.VMEM((_V_PAD,), jnp.float32),
        pltpu.VMEM((_V_PAD,), jnp.float32),
        pltpu.VMEM((_V_PAD,), jnp.float32),
        pltpu.VMEM((_CHUNK, _L_PAD), jnp.int32),
        pltpu.VMEM((_CHUNK, _LANES), jnp.float32),
        pltpu.VMEM((_LANES,), jnp.float32),
    ],
    compiler_params=pltpu.CompilerParams(needs_layout_passes=False),
)
def _bag_kernel(p_hbm, text_hbm, fcb_hbm, out_hbm,
                p0, p1, p2, idxbuf, outbuf, biasbuf):
    wid = lax.axis_index("s") * 2 + lax.axis_index("c")
    base = wid * _ROWS_PER_W

    pltpu.sync_copy(p_hbm.at[0], p0)
    pltpu.sync_copy(p_hbm.at[1], p1)
    pltpu.sync_copy(p_hbm.at[2], p2)
    pltpu.sync_copy(fcb_hbm, biasbuf)
    bvec = biasbuf[...]
    b0 = bvec[0]
    b1 = bvec[1]
    b2 = bvec[2]
    inv_l = jnp.float32(1.0 / _L)
    lane = lax.iota(jnp.int32, _LANES)

    def chunk_body(ci, carry):
        row0 = base + ci * _CHUNK
        pltpu.sync_copy(text_hbm.at[pl.ds(row0, _CHUNK)], idxbuf)

        def row_body(r, carry2):
            acc0 = jnp.zeros((_LANES,), jnp.float32)
            acc1 = jnp.zeros((_LANES,), jnp.float32)
            acc2 = jnp.zeros((_LANES,), jnp.float32)
            for j in range(_L_PAD // _LANES):
                idx = idxbuf[r, pl.ds(j * _LANES, _LANES)]
                acc0 = acc0 + plsc.load_gather(p0, [idx])
                acc1 = acc1 + plsc.load_gather(p1, [idx])
                acc2 = acc2 + plsc.load_gather(p2, [idx])
            s0 = jnp.sum(acc0) * inv_l + b0
            s1 = jnp.sum(acc1) * inv_l + b1
            s2 = jnp.sum(acc2) * inv_l + b2
            row = jnp.where(lane == 0, s0, jnp.where(lane == 1, s1, s2))
            outbuf[r, pl.ds(0, _LANES)] = row
            return carry2

        lax.fori_loop(0, _CHUNK, row_body, 0)
        pltpu.sync_copy(outbuf, out_hbm.at[pl.ds(row0, _CHUNK)])
        return carry

    lax.fori_loop(0, _NCHUNK, chunk_body, 0)


def kernel(text, emb_table, fc_w, fc_b):
    text_i32 = text.astype(jnp.int32)
    text_pad = jnp.pad(text_i32, ((0, 0), (0, _L_PAD - _L)),
                       constant_values=_NUM_EMB)
    fc_w_pad = jnp.pad(fc_w, ((0, 8 - _NUM_CLASSES), (0, 0)))
    fcb_pad = jnp.pad(fc_b, (0, _LANES - _NUM_CLASSES))
    p = _project(fc_w_pad, emb_table)
    p_pad = jnp.pad(p, ((0, 0), (0, _V_PAD - _NUM_EMB)))
    out16 = _bag_kernel(p_pad, text_pad, fcb_pad)
    return out16[:, :_NUM_CLASSES]


# speedup vs baseline: 153.1101x; 1.1514x over previous
import functools

import jax
import jax.numpy as jnp
from jax import lax
from jax.experimental import pallas as pl
from jax.experimental.pallas import tpu as pltpu
from jax.experimental.pallas import tpu_sc as plsc

_NUM_EMB = 10000
_EMB_DIM = 128
_NUM_CLASSES = 3
_B = 16384
_L = 200

_LANES = 16
_L_PAD = 208
_V_PAD = 10016
_NUM_WORKERS = 32
_ROWS_PER_W = _B // _NUM_WORKERS
_CHUNK = 64
_NCHUNK = _ROWS_PER_W // _CHUNK


def _project_body(w_ref, emb_ref, out_ref):
    out_ref[...] = lax.dot_general(
        w_ref[...], emb_ref[...], (((1,), (1,)), ((), ())),
        preferred_element_type=jnp.float32)


def _project(fc_w_pad, emb_table):
    return pl.pallas_call(
        _project_body,
        out_shape=jax.ShapeDtypeStruct((8, _NUM_EMB), jnp.float32),
    )(fc_w_pad, emb_table)


_mesh = plsc.VectorSubcoreMesh(core_axis_name="c", subcore_axis_name="s")


@functools.partial(
    pl.kernel,
    out_type=jax.ShapeDtypeStruct((_B, _LANES), jnp.float32),
    mesh=_mesh,
    scratch_types=[
        pltpu.VMEM((_V_PAD,), jnp.float32),
        pltpu.VMEM((_V_PAD,), jnp.float32),
        pltpu.VMEM((2, _CHUNK, _L_PAD), jnp.int32),
        pltpu.VMEM((_CHUNK, _LANES), jnp.float32),
        pltpu.VMEM((_LANES,), jnp.float32),
        pltpu.SemaphoreType.DMA,
        pltpu.SemaphoreType.DMA,
    ],
    compiler_params=pltpu.CompilerParams(needs_layout_passes=False),
)
def _bag_kernel(p_hbm, text_hbm, fcb_hbm, out_hbm,
                p01, p2, idxbuf, outbuf, biasbuf, sem0, sem1):
    wid = lax.axis_index("s") * 2 + lax.axis_index("c")
    base = wid * _ROWS_PER_W

    pltpu.sync_copy(p_hbm.at[0], p01)
    pltpu.sync_copy(p_hbm.at[1], p2)
    pltpu.sync_copy(fcb_hbm, biasbuf)
    bvec = biasbuf[...]
    b0 = bvec[0]
    b1 = bvec[1]
    b2 = bvec[2]
    inv_l = jnp.float32(1.0 / _L)
    lane = lax.iota(jnp.int32, _LANES)
    himask = jnp.int32(-65536)

    sems = (sem0, sem1)

    def start_idx_copy(ci, slot):
        return pltpu.async_copy(
            text_hbm.at[pl.ds(base + ci * _CHUNK, _CHUNK)],
            idxbuf.at[slot], sems[slot])

    def process_chunk(ci, slot):
        row0 = base + ci * _CHUNK

        def row_body(r, carry2):
            acc0 = jnp.zeros((_LANES,), jnp.float32)
            acc1 = jnp.zeros((_LANES,), jnp.float32)
            acc2 = jnp.zeros((_LANES,), jnp.float32)
            for j in range(_L_PAD // _LANES):
                idx = idxbuf[slot, r, pl.ds(j * _LANES, _LANES)]
                g01 = plsc.bitcast(plsc.load_gather(p01, [idx]), jnp.int32)
                acc0 = acc0 + plsc.bitcast(g01 & himask, jnp.float32)
                acc1 = acc1 + plsc.bitcast(g01 << 16, jnp.float32)
                acc2 = acc2 + plsc.load_gather(p2, [idx])
            s0 = jnp.sum(acc0) * inv_l + b0
            s1 = jnp.sum(acc1) * inv_l + b1
            s2 = jnp.sum(acc2) * inv_l + b2
            row = jnp.where(lane == 0, s0, jnp.where(lane == 1, s1, s2))
            outbuf[r, pl.ds(0, _LANES)] = row
            return carry2

        lax.fori_loop(0, _CHUNK, row_body, 0)
        pltpu.sync_copy(outbuf, out_hbm.at[pl.ds(row0, _CHUNK)])

    copies = [None, None]
    copies[0] = start_idx_copy(0, 0)
    for ci in range(_NCHUNK):
        slot = ci % 2
        if ci + 1 < _NCHUNK:
            copies[1 - slot] = start_idx_copy(ci + 1, 1 - slot)
        copies[slot].wait()
        process_chunk(ci, slot)


def _pack_tables(p):
    u0 = lax.bitcast_convert_type(p[0].astype(jnp.bfloat16), jnp.uint16)
    u1 = lax.bitcast_convert_type(p[1].astype(jnp.bfloat16), jnp.uint16)
    w01 = (u0.astype(jnp.uint32) << 16) | u1.astype(jnp.uint32)
    p01 = lax.bitcast_convert_type(w01, jnp.float32)
    return jnp.stack([p01, p[2]])


def kernel(text, emb_table, fc_w, fc_b):
    text_i32 = text.astype(jnp.int32)
    text_pad = jnp.pad(text_i32, ((0, 0), (0, _L_PAD - _L)),
                       constant_values=_NUM_EMB)
    fc_w_pad = jnp.pad(fc_w, ((0, 8 - _NUM_CLASSES), (0, 0)))
    fcb_pad = jnp.pad(fc_b, (0, _LANES - _NUM_CLASSES))
    p = _project(fc_w_pad, emb_table)
    p_pad = jnp.pad(p, ((0, 0), (0, _V_PAD - _NUM_EMB)))
    p_packed = _pack_tables(p_pad)
    out16 = _bag_kernel(p_packed, text_pad, fcb_pad)
    return out16[:, :_NUM_CLASSES]
